# baseline (speedup 1.0000x reference)
import functools
import numpy as np
import jax
import jax.numpy as jnp
from jax import lax
from jax.experimental import pallas as pl
from jax.experimental.pallas import tpu as pltpu
from jax.experimental.pallas import tpu_sc as plsc

E_EDGES = 160000
NWORKERS = 32
CHUNK = E_EDGES // NWORKERS
CPAD = CHUNK + 56
NV = CPAD // 16
EBUF = CPAD + 32
N_NODES = 10000
NPAD = N_NODES + 32
PADNODE = N_NODES
B = 32
R = 200
L = 6
LB = L * B

def _sc_hist(src_hbm, dst_hbm, et_hbm, head_hbm, tail_hbm, zt_hbm, zc_hbm,
             out_hbm, sv, dv, tv, htbl, ttbl, atbl, queue, cbuf, hv, tlv,
             sem, semb, csem):
    wid = lax.axis_index("c") * 16 + lax.axis_index("s")
    base = wid * CHUNK
    cz = pltpu.async_copy(zc_hbm, cbuf, csem)
    c3 = pltpu.async_copy(et_hbm.at[pl.ds(base, CHUNK)],
                          tv.at[pl.ds(0, CHUNK)], csem)
    z1 = pltpu.async_copy(zt_hbm, htbl, sem)
    z2 = pltpu.async_copy(zt_hbm, ttbl, sem)
    z3 = pltpu.async_copy(zt_hbm, atbl, sem)
    c1 = pltpu.async_copy(src_hbm.at[pl.ds(base, CHUNK)],
                          sv.at[pl.ds(0, CHUNK)], semb)
    c2 = pltpu.async_copy(dst_hbm.at[pl.ds(base, CHUNK)],
                          dv.at[pl.ds(0, CHUNK)], semb)
    c4 = pltpu.async_copy(head_hbm, hv, sem)
    c5 = pltpu.async_copy(tail_hbm, tlv, sem)
    with jax.named_scope("dma_tbl"):
        for c in (z1, z2, z3, c4, c5):
            c.wait()

    lanes = lax.iota(jnp.int32, 16)
    with jax.named_scope("build"):
        bits_lo = jnp.left_shift(jnp.full((16,), 1, jnp.int32), lanes)
        bits_hi = jnp.left_shift(jnp.full((16,), 1, jnp.int32), lanes + 16)
        onesi = jnp.ones((16,), jnp.int32)
        hv0 = hv[pl.ds(0, 16)]
        hv1 = hv[pl.ds(16, 16)]
        tv0 = tlv[pl.ds(0, 16)]
        tv1 = tlv[pl.ds(16, 16)]
        plsc.addupdate_scatter(htbl, [hv0], bits_lo)
        plsc.addupdate_scatter(htbl, [hv1], bits_hi)
        plsc.addupdate_scatter(ttbl, [tv0], bits_lo)
        plsc.addupdate_scatter(ttbl, [tv1], bits_hi)
        plsc.store_scatter(atbl, [hv0], onesi)
        plsc.store_scatter(atbl, [hv1], onesi)
        plsc.store_scatter(atbl, [tv0], onesi)
        plsc.store_scatter(atbl, [tv1], onesi)

    with jax.named_scope("dma_edges"):
        for c in (c1, c2):
            c.wait()
    with jax.named_scope("pad"):
        vmask = lanes < 8
        padv = jnp.full((16,), PADNODE, jnp.int32)
        sv[pl.ds(CHUNK - 8, 16)] = jnp.where(vmask, sv[pl.ds(CHUNK - 8, 16)],
                                             PADNODE)
        dv[pl.ds(CHUNK - 8, 16)] = jnp.where(vmask, dv[pl.ds(CHUNK - 8, 16)],
                                             PADNODE)
        for p in range(CHUNK + 8, CPAD, 16):
            sv[pl.ds(p, 16)] = padv
            dv[pl.ds(p, 16)] = padv

    def _scan(j, cnt):
        base4 = j * 64
        ss = [sv[pl.ds(base4 + u * 16, 16)] for u in range(4)]
        dd = [dv[pl.ds(base4 + u * 16, 16)] for u in range(4)]
        aas = [plsc.load_gather(atbl, [s]) for s in ss]
        aad = [plsc.load_gather(atbl, [d]) for d in dd]
        msks = [(aas[u] | aad[u]) != 0 for u in range(4)]
        pcs = [plsc.all_reduce_population_count(m)[0] for m in msks]
        offs = cnt
        for u in range(4):
            plsc.store_compressed(queue.at[pl.ds(offs, 16)],
                                  base4 + u * 16 + lanes, mask=msks[u])
            offs = offs + pcs[u]
        return offs

    with jax.named_scope("scan"):
        cnt = lax.fori_loop(0, NV // 4, _scan, jnp.int32(0))

    with jax.named_scope("czwait"):
        cz.wait()
        c3.wait()
    onesf = jnp.ones((16,), jnp.float32)
    queue[pl.ds(cnt, 16)] = jnp.full((16,), CHUNK, jnp.int32)

    def _proc(k, c):
        e16 = queue[pl.ds(k * 16, 16)]
        s16 = plsc.load_gather(sv, [e16])
        d16 = plsc.load_gather(dv, [e16])
        t16 = plsc.load_gather(tv, [e16])
        hs = plsc.load_gather(htbl, [s16])
        hd = plsc.load_gather(htbl, [d16])
        ts = plsc.load_gather(ttbl, [s16])
        td = plsc.load_gather(ttbl, [d16])
        m5 = hs & td
        m6 = hd & ts
        modes = [hd & ~m6, hs & ~m5, td & ~m5, ts & ~m6, m5, m6]
        u0 = hs | hd | ts | td

        def _peel(u):
            nz = u != 0
            low = u & (0 - u)
            fexp = plsc.bitcast(low.astype(jnp.float32), jnp.int32)
            b = (jnp.right_shift(fexp, 23) & 255) - 127
            b = jnp.where(nz, b, 0)
            for i in range(L):
                mi = (modes[i] & low) != 0
                plsc.addupdate_scatter(cbuf, [t16, b + (i * B)], onesf,
                                       mask=mi)
            return u & (u - 1)

        def _any(u):
            return plsc.all_reduce_population_count(u != 0)[0] > 0

        u1 = _peel(u0)

        @pl.when(_any(u1))
        def _():
            u2 = _peel(u1)

            @pl.when(_any(u2))
            def _():
                u = u2
                for _ in range(B - 2):
                    u = _peel(u)
        return c

    with jax.named_scope("proc"):
        lax.fori_loop(0, (cnt + 15) // 16, _proc, 0)
    with jax.named_scope("out"):
        pltpu.sync_copy(cbuf, out_hbm.at[wid])


def _tail_kernel(cp_ref, lab_ref, rv_ref, W1_ref, b1_ref, W2_ref, b2_ref,
                 reldW_ref, reldb_ref, concW_ref, concb_ref, fcW_ref, fcb_ref,
                 out_ref, *, n_rels, b_rows, link_mode):
    CT = jnp.sum(cp_ref[...], axis=0)
    V = (jnp.dot(rv_ref[...], W1_ref[...],
                 preferred_element_type=jnp.float32) + b1_ref[...])
    V = (jnp.dot(V, W2_ref[...],
                 preferred_element_type=jnp.float32) + b2_ref[...])
    S = lax.dot_general(CT, V, (((0,), (0,)), ((), ())),
                        preferred_element_type=jnp.float32)
    n = jnp.sum(CT, axis=0)[:, None]
    acc = jnp.zeros((b_rows, V.shape[1]), jnp.float32)
    for m in range(link_mode):
        Sm = S[m * b_rows:(m + 1) * b_rows, :]
        nm = n[m * b_rows:(m + 1) * b_rows, :]
        Tm = (jnp.dot(Sm, reldW_ref[m], preferred_element_type=jnp.float32)
              + nm * reldb_ref[m, :][None, :])
        acc = acc + Tm / (nm + 1e-30)
    rel_neighbor = acc / float(link_mode)

    lab = lab_ref[:, 0]
    loh = (lab[:, None] == jax.lax.broadcasted_iota(
        jnp.int32, (b_rows, n_rels), 1)).astype(jnp.float32)
    rel_embeds = jnp.dot(loh, V, preferred_element_type=jnp.float32)

    hcat = jnp.concatenate([rel_neighbor, rel_embeds], axis=1)
    hh = (jnp.dot(hcat, concW_ref[...], preferred_element_type=jnp.float32)
          + concb_ref[...])
    hh = jnp.maximum(hh, 0.0)
    nrm = jnp.sqrt(jnp.sum(hh * hh, axis=1, keepdims=True))
    g = hh / jnp.maximum(nrm, 1e-12)
    out_ref[...] = (jnp.dot(g, fcW_ref[...],
                            preferred_element_type=jnp.float32) + fcb_ref[...])


def kernel(src, dst, edge_type, head_ids, tail_ids, rel_labels, rel_vectors,
           W1, b1, W2, b2, reld_W, reld_b, conc_W, conc_b, fc_W, fc_b):
    DV = rel_vectors.shape[1]
    D = W1.shape[1]

    mesh = plsc.VectorSubcoreMesh(core_axis_name="c", subcore_axis_name="s")

    hist = pl.kernel(
        _sc_hist,
        mesh=mesh,
        compiler_params=pltpu.CompilerParams(needs_layout_passes=False),
        out_type=jax.ShapeDtypeStruct((NWORKERS, R, LB), jnp.float32),
        scratch_types=[
            pltpu.VMEM((EBUF,), jnp.int32),
            pltpu.VMEM((EBUF,), jnp.int32),
            pltpu.VMEM((EBUF,), jnp.int32),
            pltpu.VMEM((NPAD,), jnp.int32),
            pltpu.VMEM((NPAD,), jnp.int32),
            pltpu.VMEM((NPAD,), jnp.int32),
            pltpu.VMEM((EBUF,), jnp.int32),
            pltpu.VMEM((R, LB), jnp.float32),
            pltpu.VMEM((B,), jnp.int32),
            pltpu.VMEM((B,), jnp.int32),
            pltpu.SemaphoreType.DMA,
            pltpu.SemaphoreType.DMA,
            pltpu.SemaphoreType.DMA,
        ],
    )
    zt = jnp.zeros((NPAD,), jnp.int32)
    zc = jnp.zeros((R, LB), jnp.float32)
    cparts = hist(src, dst, edge_type, head_ids, tail_ids, zt, zc)

    def blk(shape):
        return pl.BlockSpec(shape, lambda *, _s=shape: tuple(0 for _ in _s))

    out = pl.pallas_call(
        functools.partial(_tail_kernel, n_rels=R, b_rows=B, link_mode=L),
        in_specs=[
            blk((NWORKERS, R, LB)),
            blk((B, 1)),
            blk((R, DV)),
            blk((DV, D)), blk((1, D)),
            blk((D, D)), blk((1, D)),
            blk((L, D, D)), blk((L, D)),
            blk((2 * D, D)), blk((1, D)),
            blk((D, 1)), blk((1, 1)),
        ],
        out_specs=blk((B, 1)),
        out_shape=jax.ShapeDtypeStruct((B, 1), jnp.float32),
    )(cparts, rel_labels.reshape(B, 1),
      rel_vectors, W1, b1.reshape(1, D), W2, b2.reshape(1, D),
      reld_W, reld_b, conc_W, conc_b.reshape(1, D),
      fc_W, fc_b.reshape(1, 1))
    return out


# speedup vs baseline: 1.1662x; 1.1662x over previous
import functools
import numpy as np
import jax
import jax.numpy as jnp
from jax import lax
from jax.experimental import pallas as pl
from jax.experimental.pallas import tpu as pltpu
from jax.experimental.pallas import tpu_sc as plsc

E_EDGES = 160000
NWORKERS = 32
CHUNK = E_EDGES // NWORKERS
CPAD = CHUNK + 56
NV = CPAD // 16
EBUF = CPAD + 32
N_NODES = 10000
NPAD = N_NODES + 32
PADNODE = N_NODES
B = 32
R = 200
L = 6
LB = L * B
HB = L * 16

def _sc_hist(src_hbm, dst_hbm, et_hbm, head_hbm, tail_hbm, zt_hbm, zc_hbm,
             out_hbm, sv, dv, tv, htbl, ttbl, queue, cbuf, hv, tlv,
             sem, semb, csem):
    wid = lax.axis_index("c") * 16 + lax.axis_index("s")
    base = wid * CHUNK
    cz = pltpu.async_copy(zc_hbm, cbuf, csem)
    c3 = pltpu.async_copy(et_hbm.at[pl.ds(base, CHUNK)],
                          tv.at[pl.ds(0, CHUNK)], csem)
    z1 = pltpu.async_copy(zt_hbm, htbl, sem)
    z2 = pltpu.async_copy(zt_hbm, ttbl, sem)
    c1 = pltpu.async_copy(src_hbm.at[pl.ds(base, CHUNK)],
                          sv.at[pl.ds(0, CHUNK)], semb)
    c2 = pltpu.async_copy(dst_hbm.at[pl.ds(base, CHUNK)],
                          dv.at[pl.ds(0, CHUNK)], semb)
    c4 = pltpu.async_copy(head_hbm, hv, sem)
    c5 = pltpu.async_copy(tail_hbm, tlv, sem)
    with jax.named_scope("dma_tbl"):
        for c in (z1, z2, c4, c5):
            c.wait()

    lanes = lax.iota(jnp.int32, 16)
    with jax.named_scope("build"):
        bits_lo = jnp.left_shift(jnp.full((16,), 1, jnp.int32), lanes)
        bits_hi = jnp.left_shift(jnp.full((16,), 1, jnp.int32), lanes + 16)
        hv0 = hv[pl.ds(0, 16)]
        hv1 = hv[pl.ds(16, 16)]
        tv0 = tlv[pl.ds(0, 16)]
        tv1 = tlv[pl.ds(16, 16)]
        plsc.addupdate_scatter(htbl, [hv0], bits_lo)
        plsc.addupdate_scatter(htbl, [hv1], bits_hi)
        plsc.addupdate_scatter(ttbl, [tv0], bits_lo)
        plsc.addupdate_scatter(ttbl, [tv1], bits_hi)

    with jax.named_scope("dma_edges"):
        for c in (c1, c2):
            c.wait()
    with jax.named_scope("pad"):
        vmask = lanes < 8
        padv = jnp.full((16,), PADNODE, jnp.int32)
        sv[pl.ds(CHUNK - 8, 16)] = jnp.where(vmask, sv[pl.ds(CHUNK - 8, 16)],
                                             PADNODE)
        dv[pl.ds(CHUNK - 8, 16)] = jnp.where(vmask, dv[pl.ds(CHUNK - 8, 16)],
                                             PADNODE)
        for p in range(CHUNK + 8, CPAD, 16):
            sv[pl.ds(p, 16)] = padv
            dv[pl.ds(p, 16)] = padv

    def _scan(j, cnt):
        base4 = j * 64
        ss = [sv[pl.ds(base4 + u * 16, 16)] for u in range(4)]
        dd = [dv[pl.ds(base4 + u * 16, 16)] for u in range(4)]
        hss = [plsc.load_gather(htbl, [s]) for s in ss]
        tss = [plsc.load_gather(ttbl, [s]) for s in ss]
        hdd = [plsc.load_gather(htbl, [d]) for d in dd]
        tdd = [plsc.load_gather(ttbl, [d]) for d in dd]
        msks = [(hss[u] | tss[u] | hdd[u] | tdd[u]) != 0 for u in range(4)]
        pcs = [plsc.all_reduce_population_count(m)[0] for m in msks]
        offs = cnt
        for u in range(4):
            plsc.store_compressed(queue.at[pl.ds(offs, 16)],
                                  base4 + u * 16 + lanes, mask=msks[u])
            offs = offs + pcs[u]
        return offs

    with jax.named_scope("scan"):
        cnt = lax.fori_loop(0, NV // 4, _scan, jnp.int32(0))

    with jax.named_scope("czwait"):
        cz.wait()
        c3.wait()
    queue[pl.ds(cnt, 16)] = jnp.full((16,), CHUNK, jnp.int32)

    def _proc(k, c):
        e16 = queue[pl.ds(k * 16, 16)]
        s16 = plsc.load_gather(sv, [e16])
        d16 = plsc.load_gather(dv, [e16])
        t16 = plsc.load_gather(tv, [e16])
        hs = plsc.load_gather(htbl, [s16])
        hd = plsc.load_gather(htbl, [d16])
        ts = plsc.load_gather(ttbl, [s16])
        td = plsc.load_gather(ttbl, [d16])
        m5 = hs & td
        m6 = hd & ts
        modes = [hd & ~m6, hs & ~m5, td & ~m5, ts & ~m6, m5, m6]
        u0 = hs | hd | ts | td

        def _peel(u):
            nz = u != 0
            low = u & (0 - u)
            fexp = plsc.bitcast(low.astype(jnp.float32), jnp.int32)
            b = (jnp.right_shift(fexp, 23) & 255) - 127
            b = jnp.where(nz, b, 0)
            bcol = b & 15
            bval = jnp.where(b < 16, 1, 1 << 16)
            for i in range(L):
                mi = (modes[i] & low) != 0
                plsc.addupdate_scatter(cbuf, [t16, bcol + (i * 16)], bval,
                                       mask=mi)
            return u & (u - 1)

        def _any(u):
            return plsc.all_reduce_population_count(u != 0)[0] > 0

        u1 = _peel(u0)

        @pl.when(_any(u1))
        def _():
            u2 = _peel(u1)

            @pl.when(_any(u2))
            def _():
                u = u2
                for _ in range(B - 2):
                    u = _peel(u)
        return c

    with jax.named_scope("proc"):
        lax.fori_loop(0, (cnt + 15) // 16, _proc, 0)
    with jax.named_scope("out"):
        pltpu.sync_copy(cbuf, out_hbm.at[wid])


def _tail_kernel(cp_ref, lab_ref, rv_ref, W1_ref, b1_ref, W2_ref, b2_ref,
                 reldW_ref, reldb_ref, concW_ref, concb_ref, fcW_ref, fcb_ref,
                 out_ref, *, n_rels, b_rows, link_mode):
    cp = cp_ref[...]
    lo = jnp.sum(cp & 0xFFFF, axis=0).astype(jnp.float32)
    hi = jnp.sum(jnp.right_shift(cp, 16), axis=0).astype(jnp.float32)
    V = (jnp.dot(rv_ref[...], W1_ref[...],
                 preferred_element_type=jnp.float32) + b1_ref[...])
    V = (jnp.dot(V, W2_ref[...],
                 preferred_element_type=jnp.float32) + b2_ref[...])
    S_lo = lax.dot_general(lo, V, (((0,), (0,)), ((), ())),
                           preferred_element_type=jnp.float32)
    S_hi = lax.dot_general(hi, V, (((0,), (0,)), ((), ())),
                           preferred_element_type=jnp.float32)
    n_lo = jnp.sum(lo, axis=0)[:, None]
    n_hi = jnp.sum(hi, axis=0)[:, None]
    hb = b_rows // 2
    acc = jnp.zeros((b_rows, V.shape[1]), jnp.float32)
    for m in range(link_mode):
        Sm = jnp.concatenate([S_lo[m * hb:(m + 1) * hb, :],
                              S_hi[m * hb:(m + 1) * hb, :]], axis=0)
        nm = jnp.concatenate([n_lo[m * hb:(m + 1) * hb, :],
                              n_hi[m * hb:(m + 1) * hb, :]], axis=0)
        Tm = (jnp.dot(Sm, reldW_ref[m], preferred_element_type=jnp.float32)
              + nm * reldb_ref[m, :][None, :])
        acc = acc + Tm / (nm + 1e-30)
    rel_neighbor = acc / float(link_mode)

    lab = lab_ref[:, 0]
    loh = (lab[:, None] == jax.lax.broadcasted_iota(
        jnp.int32, (b_rows, n_rels), 1)).astype(jnp.float32)
    rel_embeds = jnp.dot(loh, V, preferred_element_type=jnp.float32)

    hcat = jnp.concatenate([rel_neighbor, rel_embeds], axis=1)
    hh = (jnp.dot(hcat, concW_ref[...], preferred_element_type=jnp.float32)
          + concb_ref[...])
    hh = jnp.maximum(hh, 0.0)
    nrm = jnp.sqrt(jnp.sum(hh * hh, axis=1, keepdims=True))
    g = hh / jnp.maximum(nrm, 1e-12)
    out_ref[...] = (jnp.dot(g, fcW_ref[...],
                            preferred_element_type=jnp.float32) + fcb_ref[...])


def kernel(src, dst, edge_type, head_ids, tail_ids, rel_labels, rel_vectors,
           W1, b1, W2, b2, reld_W, reld_b, conc_W, conc_b, fc_W, fc_b):
    DV = rel_vectors.shape[1]
    D = W1.shape[1]

    mesh = plsc.VectorSubcoreMesh(core_axis_name="c", subcore_axis_name="s")

    hist = pl.kernel(
        _sc_hist,
        mesh=mesh,
        compiler_params=pltpu.CompilerParams(needs_layout_passes=False),
        out_type=jax.ShapeDtypeStruct((NWORKERS, R, HB), jnp.int32),
        scratch_types=[
            pltpu.VMEM((EBUF,), jnp.int32),
            pltpu.VMEM((EBUF,), jnp.int32),
            pltpu.VMEM((EBUF,), jnp.int32),
            pltpu.VMEM((NPAD,), jnp.int32),
            pltpu.VMEM((NPAD,), jnp.int32),
            pltpu.VMEM((EBUF,), jnp.int32),
            pltpu.VMEM((R, HB), jnp.int32),
            pltpu.VMEM((B,), jnp.int32),
            pltpu.VMEM((B,), jnp.int32),
            pltpu.SemaphoreType.DMA,
            pltpu.SemaphoreType.DMA,
            pltpu.SemaphoreType.DMA,
        ],
    )
    zt = jnp.zeros((NPAD,), jnp.int32)
    zc = jnp.zeros((R, HB), jnp.int32)
    cparts = hist(src, dst, edge_type, head_ids, tail_ids, zt, zc)

    def blk(shape):
        return pl.BlockSpec(shape, lambda *, _s=shape: tuple(0 for _ in _s))

    out = pl.pallas_call(
        functools.partial(_tail_kernel, n_rels=R, b_rows=B, link_mode=L),
        in_specs=[
            blk((NWORKERS, R, HB)),
            blk((B, 1)),
            blk((R, DV)),
            blk((DV, D)), blk((1, D)),
            blk((D, D)), blk((1, D)),
            blk((L, D, D)), blk((L, D)),
            blk((2 * D, D)), blk((1, D)),
            blk((D, 1)), blk((1, 1)),
        ],
        out_specs=blk((B, 1)),
        out_shape=jax.ShapeDtypeStruct((B, 1), jnp.float32),
    )(cparts, rel_labels.reshape(B, 1),
      rel_vectors, W1, b1.reshape(1, D), W2, b2.reshape(1, D),
      reld_W, reld_b, conc_W, conc_b.reshape(1, D),
      fc_W, fc_b.reshape(1, 1))
    return out


# speedup vs baseline: 1.3302x; 1.1406x over previous
import functools
import numpy as np
import jax
import jax.numpy as jnp
from jax import lax
from jax.experimental import pallas as pl
from jax.experimental.pallas import tpu as pltpu
from jax.experimental.pallas import tpu_sc as plsc

E_EDGES = 160000
NWORKERS = 32
CHUNK = E_EDGES // NWORKERS
CPAD = CHUNK + 56
NV = CPAD // 16
EBUF = CPAD + 32
N_NODES = 10000
NPAD = N_NODES + 32
PADNODE = N_NODES
B = 32
R = 200
L = 6
LB = L * B
HB = L * 16

def _sc_hist(src_hbm, dst_hbm, et_hbm, head_hbm, tail_hbm,
             out_hbm, sv, dv, tv, htbl, ttbl, queue, cbuf, hv, tlv,
             sem, semb, csem):
    wid = lax.axis_index("c") * 16 + lax.axis_index("s")
    base = wid * CHUNK
    c3 = pltpu.async_copy(et_hbm.at[pl.ds(base, CHUNK)],
                          tv.at[pl.ds(0, CHUNK)], csem)
    c1 = pltpu.async_copy(src_hbm.at[pl.ds(base, CHUNK)],
                          sv.at[pl.ds(0, CHUNK)], semb)
    c2 = pltpu.async_copy(dst_hbm.at[pl.ds(base, CHUNK)],
                          dv.at[pl.ds(0, CHUNK)], semb)
    c4 = pltpu.async_copy(head_hbm, hv, sem)
    c5 = pltpu.async_copy(tail_hbm, tlv, sem)

    z16 = jnp.zeros((16,), jnp.int32)

    def _ztbl(j, c):
        htbl[pl.ds(j * 16, 16)] = z16
        ttbl[pl.ds(j * 16, 16)] = z16
        return c

    with jax.named_scope("ztbl"):
        lax.fori_loop(0, NPAD // 16, _ztbl, 0)

    with jax.named_scope("dma_tbl"):
        for c in (c4, c5):
            c.wait()

    lanes = lax.iota(jnp.int32, 16)
    with jax.named_scope("build"):
        bits_lo = jnp.left_shift(jnp.full((16,), 1, jnp.int32), lanes)
        bits_hi = jnp.left_shift(jnp.full((16,), 1, jnp.int32), lanes + 16)
        hv0 = hv[pl.ds(0, 16)]
        hv1 = hv[pl.ds(16, 16)]
        tv0 = tlv[pl.ds(0, 16)]
        tv1 = tlv[pl.ds(16, 16)]
        plsc.addupdate_scatter(htbl, [hv0], bits_lo)
        plsc.addupdate_scatter(htbl, [hv1], bits_hi)
        plsc.addupdate_scatter(ttbl, [tv0], bits_lo)
        plsc.addupdate_scatter(ttbl, [tv1], bits_hi)

    def _zcb(r, c):
        for k in range(HB // 16):
            cbuf[r, pl.ds(k * 16, 16)] = z16
        return c

    with jax.named_scope("zcb"):
        lax.fori_loop(0, R, _zcb, 0)

    with jax.named_scope("dma_edges"):
        for c in (c1, c2):
            c.wait()
    with jax.named_scope("pad"):
        vmask = lanes < 8
        padv = jnp.full((16,), PADNODE, jnp.int32)
        sv[pl.ds(CHUNK - 8, 16)] = jnp.where(vmask, sv[pl.ds(CHUNK - 8, 16)],
                                             PADNODE)
        dv[pl.ds(CHUNK - 8, 16)] = jnp.where(vmask, dv[pl.ds(CHUNK - 8, 16)],
                                             PADNODE)
        for p in range(CHUNK + 8, CPAD, 16):
            sv[pl.ds(p, 16)] = padv
            dv[pl.ds(p, 16)] = padv

    def _scan(j, cnt):
        base4 = j * 64
        ss = [sv[pl.ds(base4 + u * 16, 16)] for u in range(4)]
        dd = [dv[pl.ds(base4 + u * 16, 16)] for u in range(4)]
        hss = [plsc.load_gather(htbl, [s]) for s in ss]
        tss = [plsc.load_gather(ttbl, [s]) for s in ss]
        hdd = [plsc.load_gather(htbl, [d]) for d in dd]
        tdd = [plsc.load_gather(ttbl, [d]) for d in dd]
        msks = [(hss[u] | tss[u] | hdd[u] | tdd[u]) != 0 for u in range(4)]
        pcs = [plsc.all_reduce_population_count(m)[0] for m in msks]
        offs = cnt
        for u in range(4):
            plsc.store_compressed(queue.at[pl.ds(offs, 16)],
                                  base4 + u * 16 + lanes, mask=msks[u])
            offs = offs + pcs[u]
        return offs

    with jax.named_scope("scan"):
        cnt = lax.fori_loop(0, NV // 4, _scan, jnp.int32(0))

    with jax.named_scope("czwait"):
        c3.wait()
    queue[pl.ds(cnt, 16)] = jnp.full((16,), CHUNK, jnp.int32)

    def _proc(k, c):
        e16 = queue[pl.ds(k * 16, 16)]
        s16 = plsc.load_gather(sv, [e16])
        d16 = plsc.load_gather(dv, [e16])
        t16 = plsc.load_gather(tv, [e16])
        hs = plsc.load_gather(htbl, [s16])
        hd = plsc.load_gather(htbl, [d16])
        ts = plsc.load_gather(ttbl, [s16])
        td = plsc.load_gather(ttbl, [d16])
        m5 = hs & td
        m6 = hd & ts
        modes = [hd & ~m6, hs & ~m5, td & ~m5, ts & ~m6, m5, m6]
        u0 = hs | hd | ts | td

        def _peel(u):
            nz = u != 0
            low = u & (0 - u)
            fexp = plsc.bitcast(low.astype(jnp.float32), jnp.int32)
            b = (jnp.right_shift(fexp, 23) & 255) - 127
            b = jnp.where(nz, b, 0)
            bcol = b & 15
            bval = jnp.where(b < 16, 1, 1 << 16)
            for i in range(L):
                mi = (modes[i] & low) != 0
                plsc.addupdate_scatter(cbuf, [t16, bcol + (i * 16)], bval,
                                       mask=mi)
            return u & (u - 1)

        def _any(u):
            return plsc.all_reduce_population_count(u != 0)[0] > 0

        u1 = _peel(u0)

        @pl.when(_any(u1))
        def _():
            u2 = _peel(u1)

            @pl.when(_any(u2))
            def _():
                u = u2
                for _ in range(B - 2):
                    u = _peel(u)
        return c

    with jax.named_scope("proc"):
        lax.fori_loop(0, (cnt + 15) // 16, _proc, 0)
    with jax.named_scope("out"):
        pltpu.sync_copy(cbuf, out_hbm.at[wid])


def _tail_kernel(cp_ref, lab_ref, rv_ref, W1_ref, b1_ref, W2_ref, b2_ref,
                 reldW_ref, reldb_ref, concW_ref, concb_ref, fcW_ref, fcb_ref,
                 out_ref, *, n_rels, b_rows, link_mode):
    cp = cp_ref[...]
    lo = jnp.sum(cp & 0xFFFF, axis=0).astype(jnp.float32)
    hi = jnp.sum(jnp.right_shift(cp, 16), axis=0).astype(jnp.float32)
    V = (jnp.dot(rv_ref[...], W1_ref[...],
                 preferred_element_type=jnp.float32) + b1_ref[...])
    V = (jnp.dot(V, W2_ref[...],
                 preferred_element_type=jnp.float32) + b2_ref[...])
    S_lo = lax.dot_general(lo, V, (((0,), (0,)), ((), ())),
                           preferred_element_type=jnp.float32)
    S_hi = lax.dot_general(hi, V, (((0,), (0,)), ((), ())),
                           preferred_element_type=jnp.float32)
    n_lo = jnp.sum(lo, axis=0)[:, None]
    n_hi = jnp.sum(hi, axis=0)[:, None]
    hb = b_rows // 2
    acc = jnp.zeros((b_rows, V.shape[1]), jnp.float32)
    for m in range(link_mode):
        Sm = jnp.concatenate([S_lo[m * hb:(m + 1) * hb, :],
                              S_hi[m * hb:(m + 1) * hb, :]], axis=0)
        nm = jnp.concatenate([n_lo[m * hb:(m + 1) * hb, :],
                              n_hi[m * hb:(m + 1) * hb, :]], axis=0)
        Tm = (jnp.dot(Sm, reldW_ref[m], preferred_element_type=jnp.float32)
              + nm * reldb_ref[m, :][None, :])
        acc = acc + Tm / (nm + 1e-30)
    rel_neighbor = acc / float(link_mode)

    lab = lab_ref[:, 0]
    loh = (lab[:, None] == jax.lax.broadcasted_iota(
        jnp.int32, (b_rows, n_rels), 1)).astype(jnp.float32)
    rel_embeds = jnp.dot(loh, V, preferred_element_type=jnp.float32)

    hcat = jnp.concatenate([rel_neighbor, rel_embeds], axis=1)
    hh = (jnp.dot(hcat, concW_ref[...], preferred_element_type=jnp.float32)
          + concb_ref[...])
    hh = jnp.maximum(hh, 0.0)
    nrm = jnp.sqrt(jnp.sum(hh * hh, axis=1, keepdims=True))
    g = hh / jnp.maximum(nrm, 1e-12)
    out_ref[...] = (jnp.dot(g, fcW_ref[...],
                            preferred_element_type=jnp.float32) + fcb_ref[...])


def kernel(src, dst, edge_type, head_ids, tail_ids, rel_labels, rel_vectors,
           W1, b1, W2, b2, reld_W, reld_b, conc_W, conc_b, fc_W, fc_b):
    DV = rel_vectors.shape[1]
    D = W1.shape[1]

    mesh = plsc.VectorSubcoreMesh(core_axis_name="c", subcore_axis_name="s")

    hist = pl.kernel(
        _sc_hist,
        mesh=mesh,
        compiler_params=pltpu.CompilerParams(needs_layout_passes=False),
        out_type=jax.ShapeDtypeStruct((NWORKERS, R, HB), jnp.int32),
        scratch_types=[
            pltpu.VMEM((EBUF,), jnp.int32),
            pltpu.VMEM((EBUF,), jnp.int32),
            pltpu.VMEM((EBUF,), jnp.int32),
            pltpu.VMEM((NPAD,), jnp.int32),
            pltpu.VMEM((NPAD,), jnp.int32),
            pltpu.VMEM((EBUF,), jnp.int32),
            pltpu.VMEM((R, HB), jnp.int32),
            pltpu.VMEM((B,), jnp.int32),
            pltpu.VMEM((B,), jnp.int32),
            pltpu.SemaphoreType.DMA,
            pltpu.SemaphoreType.DMA,
            pltpu.SemaphoreType.DMA,
        ],
    )
    cparts = hist(src, dst, edge_type, head_ids, tail_ids)

    def blk(shape):
        return pl.BlockSpec(shape, lambda *, _s=shape: tuple(0 for _ in _s))

    out = pl.pallas_call(
        functools.partial(_tail_kernel, n_rels=R, b_rows=B, link_mode=L),
        in_specs=[
            blk((NWORKERS, R, HB)),
            blk((B, 1)),
            blk((R, DV)),
            blk((DV, D)), blk((1, D)),
            blk((D, D)), blk((1, D)),
            blk((L, D, D)), blk((L, D)),
            blk((2 * D, D)), blk((1, D)),
            blk((D, 1)), blk((1, 1)),
        ],
        out_specs=blk((B, 1)),
        out_shape=jax.ShapeDtypeStruct((B, 1), jnp.float32),
    )(cparts, rel_labels.reshape(B, 1),
      rel_vectors, W1, b1.reshape(1, D), W2, b2.reshape(1, D),
      reld_W, reld_b, conc_W, conc_b.reshape(1, D),
      fc_W, fc_b.reshape(1, 1))
    return out


# speedup vs baseline: 1.3630x; 1.0247x over previous
import functools
import numpy as np
import jax
import jax.numpy as jnp
from jax import lax
from jax.experimental import pallas as pl
from jax.experimental.pallas import tpu as pltpu
from jax.experimental.pallas import tpu_sc as plsc

E_EDGES = 160000
NWORKERS = 32
CHUNK = E_EDGES // NWORKERS
CPAD = CHUNK + 56
NV = CPAD // 16
EBUF = CPAD + 32
N_NODES = 10000
NPAD = N_NODES + 48
PADNODE = N_NODES
B = 32
R = 200
L = 6
LB = L * B
HB = L * 16

def _sc_hist(src_hbm, dst_hbm, et_hbm, head_hbm, tail_hbm,
             out_hbm, sv, dv, tv, htbl, ttbl, queue, cbuf, hv, tlv,
             sem, semb, csem):
    wid = lax.axis_index("c") * 16 + lax.axis_index("s")
    base = wid * CHUNK
    c3 = pltpu.async_copy(et_hbm.at[pl.ds(base, CHUNK)],
                          tv.at[pl.ds(0, CHUNK)], csem)
    c1 = pltpu.async_copy(src_hbm.at[pl.ds(base, CHUNK)],
                          sv.at[pl.ds(0, CHUNK)], semb)
    c2 = pltpu.async_copy(dst_hbm.at[pl.ds(base, CHUNK)],
                          dv.at[pl.ds(0, CHUNK)], semb)
    c4 = pltpu.async_copy(head_hbm, hv, sem)
    c5 = pltpu.async_copy(tail_hbm, tlv, sem)

    z16 = jnp.zeros((16,), jnp.int32)

    def _ztbl(j, c):
        for u in range(4):
            htbl[pl.ds(j * 64 + u * 16, 16)] = z16
            ttbl[pl.ds(j * 64 + u * 16, 16)] = z16
        return c

    with jax.named_scope("ztbl"):
        lax.fori_loop(0, NPAD // 64, _ztbl, 0)

    with jax.named_scope("dma_tbl"):
        for c in (c4, c5):
            c.wait()

    lanes = lax.iota(jnp.int32, 16)
    with jax.named_scope("build"):
        bits_lo = jnp.left_shift(jnp.full((16,), 1, jnp.int32), lanes)
        bits_hi = jnp.left_shift(jnp.full((16,), 1, jnp.int32), lanes + 16)
        hv0 = hv[pl.ds(0, 16)]
        hv1 = hv[pl.ds(16, 16)]
        tv0 = tlv[pl.ds(0, 16)]
        tv1 = tlv[pl.ds(16, 16)]
        plsc.addupdate_scatter(htbl, [hv0], bits_lo)
        plsc.addupdate_scatter(htbl, [hv1], bits_hi)
        plsc.addupdate_scatter(ttbl, [tv0], bits_lo)
        plsc.addupdate_scatter(ttbl, [tv1], bits_hi)

    def _zcb(r, c):
        for v in range(4):
            for k in range(HB // 16):
                cbuf[r * 4 + v, pl.ds(k * 16, 16)] = z16
        return c

    with jax.named_scope("zcb"):
        lax.fori_loop(0, R // 4, _zcb, 0)

    with jax.named_scope("dma_edges"):
        for c in (c1, c2):
            c.wait()
    with jax.named_scope("pad"):
        vmask = lanes < 8
        padv = jnp.full((16,), PADNODE, jnp.int32)
        sv[pl.ds(CHUNK - 8, 16)] = jnp.where(vmask, sv[pl.ds(CHUNK - 8, 16)],
                                             PADNODE)
        dv[pl.ds(CHUNK - 8, 16)] = jnp.where(vmask, dv[pl.ds(CHUNK - 8, 16)],
                                             PADNODE)
        for p in range(CHUNK + 8, CPAD, 16):
            sv[pl.ds(p, 16)] = padv
            dv[pl.ds(p, 16)] = padv

    def _scan(j, cnt):
        base4 = j * 64
        ss = [sv[pl.ds(base4 + u * 16, 16)] for u in range(4)]
        dd = [dv[pl.ds(base4 + u * 16, 16)] for u in range(4)]
        hss = [plsc.load_gather(htbl, [s]) for s in ss]
        tss = [plsc.load_gather(ttbl, [s]) for s in ss]
        hdd = [plsc.load_gather(htbl, [d]) for d in dd]
        tdd = [plsc.load_gather(ttbl, [d]) for d in dd]
        msks = [(hss[u] | tss[u] | hdd[u] | tdd[u]) != 0 for u in range(4)]
        pcs = [plsc.all_reduce_population_count(m)[0] for m in msks]
        offs = cnt
        for u in range(4):
            plsc.store_compressed(queue.at[pl.ds(offs, 16)],
                                  base4 + u * 16 + lanes, mask=msks[u])
            offs = offs + pcs[u]
        return offs

    with jax.named_scope("scan"):
        cnt = lax.fori_loop(0, NV // 4, _scan, jnp.int32(0))

    with jax.named_scope("czwait"):
        c3.wait()
    queue[pl.ds(cnt, 16)] = jnp.full((16,), CHUNK, jnp.int32)

    def _proc(k, c):
        e16 = queue[pl.ds(k * 16, 16)]
        s16 = plsc.load_gather(sv, [e16])
        d16 = plsc.load_gather(dv, [e16])
        t16 = plsc.load_gather(tv, [e16])
        hs = plsc.load_gather(htbl, [s16])
        hd = plsc.load_gather(htbl, [d16])
        ts = plsc.load_gather(ttbl, [s16])
        td = plsc.load_gather(ttbl, [d16])
        m5 = hs & td
        m6 = hd & ts
        modes = [hd & ~m6, hs & ~m5, td & ~m5, ts & ~m6, m5, m6]
        u0 = hs | hd | ts | td

        def _peel(u):
            nz = u != 0
            low = u & (0 - u)
            fexp = plsc.bitcast(low.astype(jnp.float32), jnp.int32)
            b = (jnp.right_shift(fexp, 23) & 255) - 127
            b = jnp.where(nz, b, 0)
            bcol = b & 15
            bval = jnp.where(b < 16, 1, 1 << 16)
            for i in range(L):
                mi = (modes[i] & low) != 0
                plsc.addupdate_scatter(cbuf, [t16, bcol + (i * 16)], bval,
                                       mask=mi)
            return u & (u - 1)

        def _any(u):
            return plsc.all_reduce_population_count(u != 0)[0] > 0

        u1 = _peel(u0)

        @pl.when(_any(u1))
        def _():
            u2 = _peel(u1)

            @pl.when(_any(u2))
            def _():
                u = u2
                for _ in range(B - 2):
                    u = _peel(u)
        return c

    with jax.named_scope("proc"):
        lax.fori_loop(0, (cnt + 15) // 16, _proc, 0)
    with jax.named_scope("out"):
        pltpu.sync_copy(cbuf, out_hbm.at[wid])


def _tail_kernel(cp_ref, lab_ref, rv_ref, W1_ref, b1_ref, W2_ref, b2_ref,
                 reldW_ref, reldb_ref, concW_ref, concb_ref, fcW_ref, fcb_ref,
                 out_ref, *, n_rels, b_rows, link_mode):
    cp = cp_ref[...]
    lo = jnp.sum(cp & 0xFFFF, axis=0).astype(jnp.float32)
    hi = jnp.sum(jnp.right_shift(cp, 16), axis=0).astype(jnp.float32)
    V = (jnp.dot(rv_ref[...], W1_ref[...],
                 preferred_element_type=jnp.float32) + b1_ref[...])
    V = (jnp.dot(V, W2_ref[...],
                 preferred_element_type=jnp.float32) + b2_ref[...])
    S_lo = lax.dot_general(lo, V, (((0,), (0,)), ((), ())),
                           preferred_element_type=jnp.float32)
    S_hi = lax.dot_general(hi, V, (((0,), (0,)), ((), ())),
                           preferred_element_type=jnp.float32)
    n_lo = jnp.sum(lo, axis=0)[:, None]
    n_hi = jnp.sum(hi, axis=0)[:, None]
    hb = b_rows // 2
    acc = jnp.zeros((b_rows, V.shape[1]), jnp.float32)
    for m in range(link_mode):
        Sm = jnp.concatenate([S_lo[m * hb:(m + 1) * hb, :],
                              S_hi[m * hb:(m + 1) * hb, :]], axis=0)
        nm = jnp.concatenate([n_lo[m * hb:(m + 1) * hb, :],
                              n_hi[m * hb:(m + 1) * hb, :]], axis=0)
        Tm = (jnp.dot(Sm, reldW_ref[m], preferred_element_type=jnp.float32)
              + nm * reldb_ref[m, :][None, :])
        acc = acc + Tm / (nm + 1e-30)
    rel_neighbor = acc / float(link_mode)

    lab = lab_ref[:, 0]
    loh = (lab[:, None] == jax.lax.broadcasted_iota(
        jnp.int32, (b_rows, n_rels), 1)).astype(jnp.float32)
    rel_embeds = jnp.dot(loh, V, preferred_element_type=jnp.float32)

    hcat = jnp.concatenate([rel_neighbor, rel_embeds], axis=1)
    hh = (jnp.dot(hcat, concW_ref[...], preferred_element_type=jnp.float32)
          + concb_ref[...])
    hh = jnp.maximum(hh, 0.0)
    nrm = jnp.sqrt(jnp.sum(hh * hh, axis=1, keepdims=True))
    g = hh / jnp.maximum(nrm, 1e-12)
    out_ref[...] = (jnp.dot(g, fcW_ref[...],
                            preferred_element_type=jnp.float32) + fcb_ref[...])


def kernel(src, dst, edge_type, head_ids, tail_ids, rel_labels, rel_vectors,
           W1, b1, W2, b2, reld_W, reld_b, conc_W, conc_b, fc_W, fc_b):
    DV = rel_vectors.shape[1]
    D = W1.shape[1]

    mesh = plsc.VectorSubcoreMesh(core_axis_name="c", subcore_axis_name="s")

    hist = pl.kernel(
        _sc_hist,
        mesh=mesh,
        compiler_params=pltpu.CompilerParams(needs_layout_passes=False),
        out_type=jax.ShapeDtypeStruct((NWORKERS, R, HB), jnp.int32),
        scratch_types=[
            pltpu.VMEM((EBUF,), jnp.int32),
            pltpu.VMEM((EBUF,), jnp.int32),
            pltpu.VMEM((EBUF,), jnp.int32),
            pltpu.VMEM((NPAD,), jnp.int32),
            pltpu.VMEM((NPAD,), jnp.int32),
            pltpu.VMEM((EBUF,), jnp.int32),
            pltpu.VMEM((R, HB), jnp.int32),
            pltpu.VMEM((B,), jnp.int32),
            pltpu.VMEM((B,), jnp.int32),
            pltpu.SemaphoreType.DMA,
            pltpu.SemaphoreType.DMA,
            pltpu.SemaphoreType.DMA,
        ],
    )
    cparts = hist(src, dst, edge_type, head_ids, tail_ids)

    def blk(shape):
        return pl.BlockSpec(shape, lambda *, _s=shape: tuple(0 for _ in _s))

    out = pl.pallas_call(
        functools.partial(_tail_kernel, n_rels=R, b_rows=B, link_mode=L),
        in_specs=[
            blk((NWORKERS, R, HB)),
            blk((B, 1)),
            blk((R, DV)),
            blk((DV, D)), blk((1, D)),
            blk((D, D)), blk((1, D)),
            blk((L, D, D)), blk((L, D)),
            blk((2 * D, D)), blk((1, D)),
            blk((D, 1)), blk((1, 1)),
        ],
        out_specs=blk((B, 1)),
        out_shape=jax.ShapeDtypeStruct((B, 1), jnp.float32),
    )(cparts, rel_labels.reshape(B, 1),
      rel_vectors, W1, b1.reshape(1, D), W2, b2.reshape(1, D),
      reld_W, reld_b, conc_W, conc_b.reshape(1, D),
      fc_W, fc_b.reshape(1, 1))
    return out
